# baseline (speedup 1.0000x reference)
import functools

import jax
import jax.numpy as jnp
import numpy as np
from jax import lax
from jax.experimental import pallas as pl
from jax.experimental.pallas import tpu as pltpu
from jax.experimental.pallas import tpu_sc as plsc

_LIMS_F32 = np.linspace(-np.float32(np.pi), np.float32(np.pi), 20).astype(np.float32)
_COS_T = [np.float64(np.cos(np.float64(_LIMS_F32[k]))) for k in range(10, 19)]
_POS_T2 = [float(np.float32(t * t)) for t in _COS_T if t > 0]
_NEG_T2 = [float(np.float32(t * t)) for t in _COS_T if t < 0]

_NW = 32
_TR = 512
_GPT = _TR // 16


def _sc_body(x_hbm, idx_hbm, o_hbm, inbuf, idxbuf, outbuf, addrbuf, *, total, n, cpw):
    nt = cpw // _TR
    wid = lax.axis_index("s") * 2 + lax.axis_index("c")
    base = wid * cpw

    pltpu.sync_copy(x_hbm.at[pl.ds(base * 9, cpw * 9)], inbuf.at[pl.ds(0, cpw * 9)])
    pltpu.sync_copy(idx_hbm.at[pl.ds(base, cpw)], idxbuf.at[pl.ds(0, cpw)])

    @pl.when(wid != _NW - 1)
    def _():
        pltpu.sync_copy(
            x_hbm.at[pl.ds((base + cpw) * 9, 9)], inbuf.at[pl.ds(cpw * 9, 9)]
        )
        pltpu.sync_copy(
            idx_hbm.at[pl.ds(base + cpw, 1)], idxbuf.at[pl.ds(cpw, 1)]
        )

    lanes = lax.broadcasted_iota(jnp.int32, (16,), 0)
    ones = jnp.full((16,), 1.0, dtype=jnp.float32)
    zerosv = jnp.zeros((16,), dtype=jnp.float32)
    lane63 = lanes * 63

    def _zero(j, c):
        outbuf[pl.ds(j * 16, 16)] = zerosv
        return c

    lax.fori_loop(0, _TR * 63 // 16, _zero, 0)

    def gather9(rows9):
        return [plsc.load_gather(inbuf, [rows9 + c]) for c in range(9)]

    def bin_of(dot, q):
        rho = (dot * dot) / q
        neg = dot < 0.0
        acc = jnp.full((16,), 10, dtype=jnp.int32)
        for t2 in _POS_T2:
            acc = acc + (neg | (rho < t2)).astype(jnp.int32)
        for t2 in _NEG_T2:
            acc = acc + (neg & (rho > t2)).astype(jnp.int32)
        return acc

    def cosparts(a, bv, g):
        v1 = [a[i] - bv[i] for i in range(3)]
        v2 = [g[i] - bv[i] for i in range(3)]
        dot = v1[0] * v2[0] + v1[1] * v2[1] + v1[2] * v2[2]
        s1 = v1[0] * v1[0] + v1[1] * v1[1] + v1[2] * v1[2]
        s2 = v2[0] * v2[0] + v2[1] * v2[1] + v2[2] * v2[2]
        return dot, s1 * s2

    for t in range(nt):
        def group(g, c):
            r0 = t * _TR + g * 16
            rows = r0 + lanes
            rows9 = rows * 9
            cur = gather9(rows9)
            nxt = gather9(rows9 + 9)
            na, ca, cc = cur[0:3], cur[3:6], cur[6:9]
            nan_, can_ = nxt[0:3], nxt[3:6]

            d1, q1 = cosparts(na, ca, cc)
            d2, q2 = cosparts(ca, cc, nan_)
            d3, q3 = cosparts(cc, nan_, can_)

            idxv = idxbuf[pl.ds(r0, 16)]
            idxn = plsc.load_gather(idxbuf, [rows + 1])
            pos = base + rows
            good = ((idxn - idxv) == 1) & ((pos & (n - 1)) != (n - 1))

            b1 = bin_of(d1, q1)
            b2 = jnp.where(good, bin_of(d2, q2), 10) + 21
            b3 = jnp.where(good, bin_of(d3, q3), 10) + 42

            rowoff = lane63 + (g * 16 * 63)
            a1 = rowoff + b1
            a2 = rowoff + b2
            a3 = rowoff + b3
            plsc.store_scatter(outbuf, [a1], ones)
            plsc.store_scatter(outbuf, [a2], ones)
            plsc.store_scatter(outbuf, [a3], ones)
            abase = g * 48
            addrbuf[pl.ds(abase, 16)] = a1
            addrbuf[pl.ds(abase + 16, 16)] = a2
            addrbuf[pl.ds(abase + 32, 16)] = a3
            return c

        lax.fori_loop(0, _GPT, group, 0)

        grow = (base + t * _TR) * 63
        pltpu.sync_copy(outbuf, o_hbm.at[pl.ds(grow, _TR * 63)])

        def clean(j, c):
            addr = addrbuf[pl.ds(j * 16, 16)]
            plsc.store_scatter(outbuf, [addr], zerosv)
            return c

        lax.fori_loop(0, 3 * _GPT, clean, 0)


def kernel(coords, mask, residue_pdb_idx):
    del mask
    b, n = coords.shape[0], coords.shape[1]
    total = b * n
    assert total % _NW == 0
    cpw = total // _NW
    assert cpw % _TR == 0

    xflat = coords[:, :, :3, :].reshape(total * 9)
    idxflat = residue_pdb_idx.astype(jnp.int32).reshape(total)

    mesh = plsc.VectorSubcoreMesh(core_axis_name="c", subcore_axis_name="s")
    run = pl.kernel(
        functools.partial(_sc_body, total=total, n=n, cpw=cpw),
        mesh=mesh,
        compiler_params=pltpu.CompilerParams(needs_layout_passes=False),
        out_type=jax.ShapeDtypeStruct((total * 63,), jnp.float32),
        scratch_types=[
            pltpu.VMEM(((cpw + 1) * 9,), jnp.float32),
            pltpu.VMEM((cpw + 1,), jnp.int32),
            pltpu.VMEM((_TR * 63,), jnp.float32),
            pltpu.VMEM((3 * _TR,), jnp.int32),
        ],
    )
    out = run(xflat, idxflat)
    return out.reshape(b, n, 63)


# speedup vs baseline: 4.3316x; 4.3316x over previous
import functools

import jax
import jax.numpy as jnp
import numpy as np
from jax.experimental import pallas as pl

_LIMS_F32 = np.linspace(-np.float32(np.pi), np.float32(np.pi), 20).astype(np.float32)
_COS_THRESH = [float(np.float32(np.cos(np.float64(_LIMS_F32[k])))) for k in range(10, 19)]


def _body(x_ref, idx_ref, o_ref, *, n, s):
    def comp(j):
        return x_ref[j, 0]

    def nxtflat(a):
        col0 = a[:, 0:1]
        col0s = jnp.concatenate([col0[1:], col0[:1]], axis=0)
        return jnp.concatenate([a[:, 1:], col0s], axis=1)

    nx, ny, nz = comp(0), comp(1), comp(2)
    cax, cay, caz = comp(3), comp(4), comp(5)
    cx, cy, cz = comp(6), comp(7), comp(8)
    nxn, nyn, nzn = nxtflat(nx), nxtflat(ny), nxtflat(nz)
    caxn, cayn, cazn = nxtflat(cax), nxtflat(cay), nxtflat(caz)

    def cos_angle(ax, ay, az, bx, by, bz, gx, gy, gz):
        v1x, v1y, v1z = ax - bx, ay - by, az - bz
        v2x, v2y, v2z = gx - bx, gy - by, gz - bz
        dot = v1x * v2x + v1y * v2y + v1z * v2z
        s1 = v1x * v1x + v1y * v1y + v1z * v1z
        s2 = v2x * v2x + v2y * v2y + v2z * v2z
        return dot / (jnp.sqrt(s1) * jnp.sqrt(s2) + 1e-10)

    cos1 = cos_angle(nx, ny, nz, cax, cay, caz, cx, cy, cz)
    cos2 = cos_angle(cax, cay, caz, cx, cy, cz, nxn, nyn, nzn)
    cos3 = cos_angle(cx, cy, cz, nxn, nyn, nzn, caxn, cayn, cazn)

    idx = idx_ref[0, 0]
    sub = jax.lax.broadcasted_iota(jnp.int32, (s, 128), 0)
    lane = jax.lax.broadcasted_iota(jnp.int32, (s, 128), 1)
    good = ((nxtflat(idx) - idx) == 1) & ~((sub == s - 1) & (lane == 127))
    cos2 = jnp.where(good, cos2, 2.0)
    cos3 = jnp.where(good, cos3, 2.0)

    def bin_of(c):
        acc = jnp.full((s, 128), 10, dtype=jnp.int32)
        for t in _COS_THRESH:
            acc = acc + (c < t).astype(jnp.int32)
        return acc

    b1, b2, b3 = bin_of(cos1), bin_of(cos2), bin_of(cos3)

    zero = jnp.zeros((s, 128), dtype=jnp.float32)
    for c in range(63):
        if 10 <= c <= 19:
            v = (b1 == c).astype(jnp.float32)
        elif 31 <= c <= 40:
            v = (b2 == c - 21).astype(jnp.float32)
        elif 52 <= c <= 61:
            v = (b3 == c - 42).astype(jnp.float32)
        else:
            v = zero
        o_ref[c, 0] = v


def kernel(coords, mask, residue_pdb_idx):
    del mask
    b, n = coords.shape[0], coords.shape[1]
    assert n % 128 == 0
    s = n // 128
    nat3 = coords.shape[2] * coords.shape[3]
    xt = (
        jnp.transpose(coords, (2, 3, 0, 1))
        .reshape(nat3, b, n)[:9]
        .reshape(9, b, s, 128)
    )
    idxt = residue_pdb_idx.astype(jnp.int32).reshape(b, 1, s, 128)

    ot = pl.pallas_call(
        functools.partial(_body, n=n, s=s),
        grid=(b,),
        in_specs=[
            pl.BlockSpec((9, 1, s, 128), lambda i: (0, i, 0, 0)),
            pl.BlockSpec((1, 1, s, 128), lambda i: (i, 0, 0, 0)),
        ],
        out_specs=pl.BlockSpec((63, 1, s, 128), lambda i: (0, i, 0, 0)),
        out_shape=jax.ShapeDtypeStruct((63, b, s, 128), jnp.float32),
    )(xt, idxt)
    return ot.reshape(63, b, n).transpose(1, 2, 0)
